# baseline (speedup 1.0000x reference)
import functools

import jax
import jax.numpy as jnp
from jax import lax
from jax.experimental import pallas as pl
from jax.experimental.pallas import tpu as pltpu
from jax.experimental.pallas import tpu_sc as plsc

B = 4096
HIST = 200
D = 64

_NC, _NS = 2, 16
_NW = _NC * _NS
_CH = 128
_NB = 2
_SETS = 2
_SG = _CH * _NB * _SETS

_NQ = B * HIST // _NW
_NT = B * D // _NW


def _gather_stream(tab, idx_v, out, row0, chunk0, nsuper, bufs, gsem, wsem):
    def body(g, carry):
        base = g * _SG
        for s in range(_SETS):
            sbase = base + s * _NB * _CH

            @pl.when(g > 0)
            def _():
                for b in range(_NB):
                    pltpu.make_async_copy(
                        bufs.at[s].at[b],
                        out.at[pl.ds(row0, _CH)],
                        wsem.at[s],
                    ).wait()

            handles = []
            for b in range(_NB):
                lc = chunk0 + g * (_SETS * _NB) + s * _NB + b
                h = pltpu.make_async_copy(
                    tab.at[idx_v.at[lc]], bufs.at[s].at[b], gsem.at[s])
                h.start()
                handles.append(h)
            for h in handles:
                h.wait()
            for b in range(_NB):
                crow = row0 + sbase + b * _CH
                pltpu.make_async_copy(
                    bufs.at[s].at[b], out.at[pl.ds(crow, _CH)], wsem.at[s]
                ).start()
        return carry

    lax.fori_loop(0, nsuper, body, 0)
    for s in range(_SETS):
        for b in range(_NB):
            pltpu.make_async_copy(
                bufs.at[s].at[b], out.at[pl.ds(row0, _CH)], wsem.at[s]
            ).wait()


def _sc_gather_body(nrows, stride, half, idx2d, tab, out,
                    idx_v, bufs, gsem, wsem):
    wid = lax.axis_index("s") * _NC + lax.axis_index("c")
    nc = nrows // _CH
    nload = (nc + 23) // 8 * 8
    start = half * nc * _NW + wid * nc
    base = pl.multiple_of(
        jnp.minimum(start - start % 8, stride * _NW - nload), 8)
    skip = start - base
    pltpu.sync_copy(idx2d.at[pl.ds(base, nload)],
                    idx_v.at[pl.ds(0, nload)])
    _gather_stream(tab, idx_v, out, wid * nrows, skip, nrows // _SG,
                   bufs, gsem, wsem)


def _sc_gather(idx2d, tab, total_rows, stride, half=0):
    nrows = total_rows // _NW
    mesh = plsc.VectorSubcoreMesh(core_axis_name="c", subcore_axis_name="s")
    return pl.kernel(
        functools.partial(_sc_gather_body, nrows, stride, half),
        out_type=jax.ShapeDtypeStruct((total_rows, 128), jnp.float32),
        mesh=mesh,
        compiler_params=pltpu.CompilerParams(use_tc_tiling_on_sc=True),
        scratch_types=[
            pltpu.VMEM(((nrows // _CH + 23) // 8 * 8, _CH), jnp.int32),
            pltpu.VMEM((_SETS, _NB, _CH, 128), jnp.float32),
            pltpu.SemaphoreType.DMA((_SETS,)),
            pltpu.SemaphoreType.DMA((_SETS,)),
        ],
    )(idx2d, tab)


_G = 16


def _bmm_body(q_ref, t_ref, o_ref):
    for i in range(_G):
        qv = q_ref[pl.ds(i * HIST, HIST), :D].astype(jnp.bfloat16)
        tv = t_ref[pl.ds(i * D, D), :D].astype(jnp.bfloat16)
        o_ref[i] = jnp.dot(qv, tv, preferred_element_type=jnp.float32)


def _tc_bmm(q2, t2, nb, t_goff):
    return pl.pallas_call(
        _bmm_body,
        grid=(nb // _G,),
        in_specs=[
            pl.BlockSpec((_G * HIST, 128), lambda g: (g, 0)),
            pl.BlockSpec((_G * D, 128), lambda g: (g + t_goff, 0)),
        ],
        out_specs=pl.BlockSpec((_G, HIST, D), lambda g: (g, 0, 0)),
        out_shape=jax.ShapeDtypeStruct((nb, HIST, D), jnp.float32),
    )(q2, t2)


def kernel(user, item_i, item_j, user_bias_table, item_bias_table,
           query_table, target_table):
    qidx = item_j.reshape(-1, _CH)
    tidx = item_i.reshape(-1, _CH)
    qt = jnp.pad(query_table, ((0, 0), (0, 128 - D)))
    tt = jnp.pad(target_table, ((0, 0), (0, 128 - D)))
    t_gath = _sc_gather(tidx, tt, B * D, B * D // _NW // _CH)
    half = B * HIST // 2
    stride = B * HIST // _NW // _CH
    q_a = _sc_gather(qidx, qt, half, stride, 0)
    q_b = _sc_gather(qidx, qt, half, stride, 1)
    p_a = _tc_bmm(q_a, t_gath, B // 2, 0)
    p_b = _tc_bmm(q_b, t_gath, B // 2, B // 2 // _G)
    return jnp.concatenate([p_a, p_b], axis=0)


# speedup vs baseline: 1.0628x; 1.0628x over previous
import functools

import jax
import jax.numpy as jnp
from jax import lax
from jax.experimental import pallas as pl
from jax.experimental.pallas import tpu as pltpu
from jax.experimental.pallas import tpu_sc as plsc

B = 4096
HIST = 200
D = 64

_NC, _NS = 2, 16
_NW = _NC * _NS
_CH = 128
_NB = 2
_SETS = 2
_SG = _CH * _NB * _SETS

_NQ = B * HIST // _NW
_NT = B * D // _NW


def _gather_stream(tab, idx_v, out, row0, chunk0, nsuper, bufs, gsem, wsem):
    def body(g, carry):
        base = g * _SG
        for s in range(_SETS):
            sbase = base + s * _NB * _CH

            @pl.when(g > 0)
            def _():
                for b in range(_NB):
                    pltpu.make_async_copy(
                        bufs.at[s].at[b],
                        out.at[pl.ds(row0, _CH)],
                        wsem.at[s],
                    ).wait()

            handles = []
            for b in range(_NB):
                lc = chunk0 + g * (_SETS * _NB) + s * _NB + b
                h = pltpu.make_async_copy(
                    tab.at[idx_v.at[lc]], bufs.at[s].at[b], gsem.at[s])
                h.start()
                handles.append(h)
            for h in handles:
                h.wait()
            for b in range(_NB):
                crow = row0 + sbase + b * _CH
                pltpu.make_async_copy(
                    bufs.at[s].at[b], out.at[pl.ds(crow, _CH)], wsem.at[s]
                ).start()
        return carry

    lax.fori_loop(0, nsuper, body, 0)
    for s in range(_SETS):
        for b in range(_NB):
            pltpu.make_async_copy(
                bufs.at[s].at[b], out.at[pl.ds(row0, _CH)], wsem.at[s]
            ).wait()


def _sc_gather_body(nrows, stride, half, idx2d, tab, out,
                    idx_v, bufs, gsem, wsem):
    wid = lax.axis_index("s") * _NC + lax.axis_index("c")
    nc = nrows // _CH
    nload = (nc + 23) // 8 * 8
    start = half * nc * _NW + wid * nc
    base = pl.multiple_of(
        jnp.minimum(start - start % 8, stride * _NW - nload), 8)
    skip = start - base
    pltpu.sync_copy(idx2d.at[pl.ds(base, nload)],
                    idx_v.at[pl.ds(0, nload)])
    _gather_stream(tab, idx_v, out, wid * nrows, skip, nrows // _SG,
                   bufs, gsem, wsem)


def _sc_gather(idx2d, tab, total_rows, stride, half=0):
    nrows = total_rows // _NW
    mesh = plsc.VectorSubcoreMesh(core_axis_name="c", subcore_axis_name="s")
    return pl.kernel(
        functools.partial(_sc_gather_body, nrows, stride, half),
        out_type=jax.ShapeDtypeStruct((total_rows, 128), jnp.float32),
        mesh=mesh,
        compiler_params=pltpu.CompilerParams(use_tc_tiling_on_sc=True),
        scratch_types=[
            pltpu.VMEM(((nrows // _CH + 23) // 8 * 8, _CH), jnp.int32),
            pltpu.VMEM((_SETS, _NB, _CH, 128), jnp.float32),
            pltpu.SemaphoreType.DMA((_SETS,)),
            pltpu.SemaphoreType.DMA((_SETS,)),
        ],
    )(idx2d, tab)


_G = 16


def _bmm_body(q_ref, t_ref, o_ref):
    for i in range(_G):
        qv = q_ref[pl.ds(i * HIST, HIST), :D].astype(jnp.bfloat16)
        tv = t_ref[pl.ds(i * D, D), :D].astype(jnp.bfloat16)
        o_ref[i] = jnp.dot(qv, tv, preferred_element_type=jnp.float32)


def _tc_bmm(q2, t2, nb, t_goff):
    return pl.pallas_call(
        _bmm_body,
        grid=(nb // _G,),
        in_specs=[
            pl.BlockSpec((_G * HIST, 128), lambda g: (g, 0)),
            pl.BlockSpec((_G * D, 128), lambda g: (g + t_goff, 0)),
        ],
        out_specs=pl.BlockSpec((_G, HIST, D), lambda g: (g, 0, 0)),
        out_shape=jax.ShapeDtypeStruct((nb, HIST, D), jnp.float32),
    )(q2, t2)


def kernel(user, item_i, item_j, user_bias_table, item_bias_table,
           query_table, target_table):
    qidx = item_j.reshape(-1, _CH)
    tidx = item_i.reshape(-1, _CH)
    qt = jnp.pad(query_table, ((0, 0), (0, 128 - D)))
    tt = jnp.pad(target_table, ((0, 0), (0, 128 - D)))
    q_gath = _sc_gather(qidx, qt, B * HIST, B * HIST // _NW // _CH)
    t_gath = _sc_gather(tidx, tt, B * D, B * D // _NW // _CH)
    return _tc_bmm(q_gath, t_gath, B, 0)


# speedup vs baseline: 1.1011x; 1.0360x over previous
import functools

import jax
import jax.numpy as jnp
from jax import lax
from jax.experimental import pallas as pl
from jax.experimental.pallas import tpu as pltpu
from jax.experimental.pallas import tpu_sc as plsc

B = 4096
HIST = 200
D = 64

_NC, _NS = 2, 16
_NW = _NC * _NS
_CH = 128
_NB = 2
_SETS = 2
_SG = _CH * _NB * _SETS

_NQ = B * HIST // _NW
_NT = B * D // _NW


def _gather_stream(tab, idx_v, out, row0, chunk0, nsuper, bufs, gsem, wsem):
    def body(g, carry):
        base = g * _SG
        for s in range(_SETS):
            sbase = base + s * _NB * _CH

            @pl.when(g > 0)
            def _():
                for b in range(_NB):
                    pltpu.make_async_copy(
                        bufs.at[s].at[b],
                        out.at[pl.ds(row0, _CH)],
                        wsem.at[s],
                    ).wait()

            handles = []
            for b in range(_NB):
                lc = chunk0 + g * (_SETS * _NB) + s * _NB + b
                h = pltpu.make_async_copy(
                    tab.at[idx_v.at[lc]], bufs.at[s].at[b], gsem.at[s])
                h.start()
                handles.append(h)
            for h in handles:
                h.wait()
            for b in range(_NB):
                crow = row0 + sbase + b * _CH
                pltpu.make_async_copy(
                    bufs.at[s].at[b], out.at[pl.ds(crow, _CH)], wsem.at[s]
                ).start()
        return carry

    lax.fori_loop(0, nsuper, body, 0)
    for s in range(_SETS):
        for b in range(_NB):
            pltpu.make_async_copy(
                bufs.at[s].at[b], out.at[pl.ds(row0, _CH)], wsem.at[s]
            ).wait()


def _sc_gather_body(nrows, stride, half, idx2d, tab, out,
                    idx_v, bufs, gsem, wsem):
    wid = lax.axis_index("s") * _NC + lax.axis_index("c")
    nc = nrows // _CH
    nload = (nc + 23) // 8 * 8
    start = half * nc * _NW + wid * nc
    base = pl.multiple_of(
        jnp.minimum(start - start % 8, stride * _NW - nload), 8)
    skip = start - base
    pltpu.sync_copy(idx2d.at[pl.ds(base, nload)],
                    idx_v.at[pl.ds(0, nload)])
    _gather_stream(tab, idx_v, out, wid * nrows, skip, nrows // _SG,
                   bufs, gsem, wsem)


def _sc_gather(idx2d, tab, total_rows, stride, half=0):
    nrows = total_rows // _NW
    mesh = plsc.VectorSubcoreMesh(core_axis_name="c", subcore_axis_name="s")
    return pl.kernel(
        functools.partial(_sc_gather_body, nrows, stride, half),
        out_type=jax.ShapeDtypeStruct((total_rows, 128), jnp.float32),
        mesh=mesh,
        compiler_params=pltpu.CompilerParams(use_tc_tiling_on_sc=True),
        scratch_types=[
            pltpu.VMEM(((nrows // _CH + 23) // 8 * 8, _CH), jnp.int32),
            pltpu.VMEM((_SETS, _NB, _CH, 128), jnp.float32),
            pltpu.SemaphoreType.DMA((_SETS,)),
            pltpu.SemaphoreType.DMA((_SETS,)),
        ],
    )(idx2d, tab)


_G = 32


def _bmm_body(q_ref, t_ref, o_ref):
    for i in range(_G):
        qv = q_ref[pl.ds(i * HIST, HIST), :D].astype(jnp.bfloat16)
        tv = t_ref[pl.ds(i * D, D), :D].astype(jnp.bfloat16)
        o_ref[i] = jnp.dot(qv, tv, preferred_element_type=jnp.float32)


def _tc_bmm(q2, t2, nb, t_goff):
    return pl.pallas_call(
        _bmm_body,
        grid=(nb // _G,),
        in_specs=[
            pl.BlockSpec((_G * HIST, 128), lambda g: (g, 0)),
            pl.BlockSpec((_G * D, 128), lambda g: (g + t_goff, 0)),
        ],
        out_specs=pl.BlockSpec((_G, HIST, D), lambda g: (g, 0, 0)),
        out_shape=jax.ShapeDtypeStruct((nb, HIST, D), jnp.float32),
    )(q2, t2)


def kernel(user, item_i, item_j, user_bias_table, item_bias_table,
           query_table, target_table):
    qidx = item_j.reshape(-1, _CH)
    tidx = item_i.reshape(-1, _CH)
    qt = jnp.pad(query_table, ((0, 0), (0, 128 - D)))
    tt = jnp.pad(target_table, ((0, 0), (0, 128 - D)))
    q_gath = _sc_gather(qidx, qt, B * HIST, B * HIST // _NW // _CH)
    t_gath = _sc_gather(tidx, tt, B * D, B * D // _NW // _CH)
    return _tc_bmm(q_gath, t_gath, B, 0)
